# Initial kernel scaffold; baseline (speedup 1.0000x reference)
import jax
import jax.numpy as jnp
from jax.experimental import pallas as pl


def kernel(x, edge_index, W_enc, b_enc, W_g0, b_g0, W_g1, b_g1, W_g2, b_g2, W_p1, b_p1, W_p2, b_p2):
    raise NotImplementedError("write your pallas kernel here")


# speedup vs baseline: 10.3786x; 10.3786x over previous
import jax
import jax.numpy as jnp
from jax import lax
from jax.experimental import pallas as pl
from jax.experimental.pallas import tpu as pltpu
from jax.experimental.pallas import tpu_sc as plsc

N = 10000
E = 320000
D = 128

NC = 2
NS = 16
NW = NC * NS
CH = 79
EPAD = NW * CH * 128
TBL = 10240
STRIPE = TBL // NS
KS = STRIPE // 128
DUMMY = N

BM = 2000
G = N // BM

_mesh = plsc.VectorSubcoreMesh(core_axis_name="c", subcore_axis_name="s")


def _deg_body(dst3, ones128, zeros128, out, dstv, onesv, table):
    c = lax.axis_index("c")
    s = lax.axis_index("s")
    wid = c * NS + s
    base = s * STRIPE
    for k in range(KS):
        pltpu.sync_copy(zeros128, table.at[pl.ds(base + k * 128, 128)])
    pltpu.sync_copy(dst3.at[wid], dstv)
    pltpu.sync_copy(ones128, onesv)
    plsc.subcore_barrier()

    def body(j, carry):
        pltpu.sync_copy(onesv, table.at[dstv.at[j]], add=True)
        return carry

    lax.fori_loop(0, CH, body, 0)
    plsc.subcore_barrier()
    for k in range(KS):
        pltpu.sync_copy(table.at[pl.ds(base + k * 128, 128)],
                        out.at[c, pl.ds(base + k * 128, 128)])


_deg = pl.kernel(
    _deg_body,
    mesh=_mesh,
    out_type=jax.ShapeDtypeStruct((NC, TBL, D), jnp.float32),
    scratch_types=[
        pltpu.VMEM((CH, 128), jnp.int32),
        pltpu.VMEM((128, D), jnp.float32),
        pltpu.VMEM_SHARED((TBL, D), jnp.float32),
    ],
)


def _layer_body(mp, src3, dst3, zeros128, out, srcv, dstv, rows, table, sem):
    c = lax.axis_index("c")
    s = lax.axis_index("s")
    wid = c * NS + s
    base = s * STRIPE
    for k in range(KS):
        pltpu.sync_copy(zeros128, table.at[pl.ds(base + k * 128, 128)])
    pltpu.sync_copy(src3.at[wid], srcv)
    pltpu.sync_copy(dst3.at[wid], dstv)
    plsc.subcore_barrier()

    def body(j, carry):
        pltpu.async_copy(mp.at[srcv.at[j]], rows, sem).wait()
        pltpu.sync_copy(rows, table.at[dstv.at[j]], add=True)
        return carry

    lax.fori_loop(0, CH, body, 0)
    plsc.subcore_barrier()
    for k in range(KS):
        pltpu.sync_copy(table.at[pl.ds(base + k * 128, 128)],
                        out.at[c, pl.ds(base + k * 128, 128)])


_layer = pl.kernel(
    _layer_body,
    mesh=_mesh,
    out_type=jax.ShapeDtypeStruct((NC, TBL, D), jnp.float32),
    scratch_types=[
        pltpu.VMEM((CH, 128), jnp.int32),
        pltpu.VMEM((CH, 128), jnp.int32),
        pltpu.VMEM((128, D), jnp.float32),
        pltpu.VMEM_SHARED((TBL, D), jnp.float32),
        pltpu.SemaphoreType.DMA,
    ],
)


def _enc_body(x_ref, we_ref, be_ref, wg_ref, out_ref):
    h = jnp.maximum(
        jnp.dot(x_ref[...], we_ref[...], preferred_element_type=jnp.float32)
        + be_ref[...], 0.0)
    out_ref[...] = jnp.dot(h, wg_ref[...], preferred_element_type=jnp.float32)


_enc = pl.pallas_call(
    _enc_body,
    grid=(G,),
    in_specs=[
        pl.BlockSpec((BM, D), lambda i: (i, 0)),
        pl.BlockSpec((D, D), lambda i: (0, 0)),
        pl.BlockSpec((1, D), lambda i: (0, 0)),
        pl.BlockSpec((D, D), lambda i: (0, 0)),
    ],
    out_specs=pl.BlockSpec((BM, D), lambda i: (i, 0)),
    out_shape=jax.ShapeDtypeStruct((N, D), jnp.float32),
)


def _scale_body(hm_ref, cnt_ref, mp_ref, dis_ref):
    deg = 1.0 + cnt_ref[0, :, 0:1] + cnt_ref[1, :, 0:1]
    dis = lax.rsqrt(deg)
    dis_ref[...] = jnp.broadcast_to(dis, (BM, D))
    mp_ref[...] = hm_ref[...] * dis


_scale = pl.pallas_call(
    _scale_body,
    grid=(G,),
    in_specs=[
        pl.BlockSpec((BM, D), lambda i: (i, 0)),
        pl.BlockSpec((NC, BM, D), lambda i: (0, i, 0)),
    ],
    out_specs=[
        pl.BlockSpec((BM, D), lambda i: (i, 0)),
        pl.BlockSpec((BM, D), lambda i: (i, 0)),
    ],
    out_shape=[
        jax.ShapeDtypeStruct((N, D), jnp.float32),
        jax.ShapeDtypeStruct((N, D), jnp.float32),
    ],
)


def _combine_body(parts_ref, mp_ref, dis_ref, b_ref, w_ref, out_ref):
    dis = dis_ref[...]
    q = parts_ref[0] + parts_ref[1] + mp_ref[...]
    h = jnp.maximum(dis * q + b_ref[...], 0.0)
    out_ref[...] = jnp.dot(
        h, w_ref[...], preferred_element_type=jnp.float32) * dis


_combine = pl.pallas_call(
    _combine_body,
    grid=(G,),
    in_specs=[
        pl.BlockSpec((NC, BM, D), lambda i: (0, i, 0)),
        pl.BlockSpec((BM, D), lambda i: (i, 0)),
        pl.BlockSpec((BM, D), lambda i: (i, 0)),
        pl.BlockSpec((1, D), lambda i: (0, 0)),
        pl.BlockSpec((D, D), lambda i: (0, 0)),
    ],
    out_specs=pl.BlockSpec((BM, D), lambda i: (i, 0)),
    out_shape=jax.ShapeDtypeStruct((N, D), jnp.float32),
)


def _final_body(parts_ref, mp_ref, dis_ref, b_ref, wp1_ref, bp1_ref,
                wp2_ref, bp2_ref, out_ref, acc_ref):
    i = pl.program_id(0)

    @pl.when(i == 0)
    def _():
        acc_ref[...] = jnp.zeros_like(acc_ref)

    h = jnp.maximum(
        dis_ref[...] * (parts_ref[0] + parts_ref[1] + mp_ref[...])
        + b_ref[...], 0.0)
    acc_ref[...] += jnp.sum(h, axis=0, keepdims=True)

    @pl.when(i == G - 1)
    def _():
        g = acc_ref[...] * (1.0 / N)
        z = jnp.maximum(
            jnp.dot(g, wp1_ref[...], preferred_element_type=jnp.float32)
            + bp1_ref[...], 0.0)
        out_ref[...] = jnp.dot(
            z, wp2_ref[...], preferred_element_type=jnp.float32) + bp2_ref[...]


_final = pl.pallas_call(
    _final_body,
    grid=(G,),
    in_specs=[
        pl.BlockSpec((NC, BM, D), lambda i: (0, i, 0)),
        pl.BlockSpec((BM, D), lambda i: (i, 0)),
        pl.BlockSpec((BM, D), lambda i: (i, 0)),
        pl.BlockSpec((1, D), lambda i: (0, 0)),
        pl.BlockSpec((D, D // 2), lambda i: (0, 0)),
        pl.BlockSpec((1, D // 2), lambda i: (0, 0)),
        pl.BlockSpec((D // 2, D), lambda i: (0, 0)),
        pl.BlockSpec((1, D), lambda i: (0, 0)),
    ],
    out_specs=pl.BlockSpec((1, D), lambda i: (0, 0)),
    out_shape=jax.ShapeDtypeStruct((1, D), jnp.float32),
    scratch_shapes=[pltpu.VMEM((1, D), jnp.float32)],
)


def kernel(x, edge_index, W_enc, b_enc, W_g0, b_g0, W_g1, b_g1, W_g2, b_g2,
           W_p1, b_p1, W_p2, b_p2):
    src = edge_index[0]
    dst = edge_index[1]
    pad = EPAD - E
    src3 = jnp.concatenate(
        [src, jnp.zeros((pad,), jnp.int32)]).reshape(NW, CH, 128)
    dst3 = jnp.concatenate(
        [dst, jnp.full((pad,), DUMMY, jnp.int32)]).reshape(NW, CH, 128)
    ones128 = jnp.ones((128, D), jnp.float32)
    zeros128 = jnp.zeros((128, D), jnp.float32)

    counts = _deg(dst3, ones128, zeros128)
    h0m = _enc(x, W_enc, b_enc.reshape(1, D), W_g0)
    mp, disb = _scale(h0m, counts)
    parts = _layer(mp, src3, dst3, zeros128)
    mp = _combine(parts, mp, disb, b_g0.reshape(1, D), W_g1)
    parts = _layer(mp, src3, dst3, zeros128)
    mp = _combine(parts, mp, disb, b_g1.reshape(1, D), W_g2)
    parts = _layer(mp, src3, dst3, zeros128)
    out = _final(parts, mp, disb, b_g2.reshape(1, D), W_p1,
                 b_p1.reshape(1, D // 2), W_p2, b_p2.reshape(1, D))
    return out.reshape(D)
